# baseline (speedup 1.0000x reference)
import functools

import jax
import jax.numpy as jnp
from jax.experimental import pallas as pl
from jax.experimental.pallas import tpu as pltpu

E = 8
TOPK = 2
D_MODEL = 1024
D_FF = 4096
N = 2048
C = 512


def _router_body(tok_ref, wg_ref, flat1_ref, flat2_ref, g1_ref, g2_ref):
    tok = tok_ref[...]
    wg = wg_ref[...]
    logits = jnp.dot(tok, wg, preferred_element_type=jnp.float32)
    m = jnp.max(logits, axis=1, keepdims=True)
    ex = jnp.exp(logits - m)
    probs = ex / jnp.sum(ex, axis=1, keepdims=True)

    col = jax.lax.broadcasted_iota(jnp.int32, (N, E), 1)
    big = jnp.int32(E)
    m1 = jnp.max(probs, axis=1, keepdims=True)
    a1 = jnp.min(jnp.where(probs == m1, col, big), axis=1, keepdims=True)
    p2 = jnp.where(col == a1, -1.0, probs)
    m2 = jnp.max(p2, axis=1, keepdims=True)
    a2 = jnp.min(jnp.where(p2 == m2, col, big), axis=1, keepdims=True)

    mask1 = (col == a1).astype(jnp.float32)
    mask2 = (col == a2).astype(jnp.float32)

    ri = jax.lax.broadcasted_iota(jnp.int32, (N, N), 0)
    ci = jax.lax.broadcasted_iota(jnp.int32, (N, N), 1)
    tril = (ci <= ri).astype(jnp.float32)
    cum1 = jnp.dot(tril, mask1, preferred_element_type=jnp.float32)
    cum2 = jnp.dot(tril, mask2, preferred_element_type=jnp.float32)

    pos1 = jnp.sum(cum1 * mask1, axis=1, keepdims=True) - 1.0
    c1 = jnp.sum(mask1, axis=0, keepdims=True)
    pos2 = (jnp.sum(cum2 * mask2, axis=1, keepdims=True) - 1.0
            + jnp.sum(c1 * mask2, axis=1, keepdims=True))
    pos1i = pos1.astype(jnp.int32)
    pos2i = pos2.astype(jnp.int32)

    keep1 = pos1i < C
    keep2 = pos2i < C
    flat1_ref[...] = jnp.where(keep1, a1 * C + pos1i, E * C)
    flat2_ref[...] = jnp.where(keep2, a2 * C + pos2i, E * C)
    g1 = jnp.where(keep1, m1, 0.0)
    g2 = jnp.where(keep2, m2, 0.0)
    denom = g1 + g2 + 1e-9
    g1_ref[...] = g1 / denom
    g2_ref[...] = g2 / denom


def _ffn_body(tok_ref, flat1_ref, flat2_ref, w1_ref, b1_ref, w2_ref, b2_ref,
              out_ref):
    e = pl.program_id(0)
    slot = jax.lax.broadcasted_iota(jnp.int32, (C, N), 0) + e * C
    f1 = flat1_ref[...]
    f2 = flat2_ref[...]
    p = ((slot == f1) | (slot == f2)).astype(jnp.bfloat16)
    eb = jnp.dot(p, tok_ref[...], preferred_element_type=jnp.float32)

    h = jnp.dot(eb.astype(jnp.bfloat16), w1_ref[0],
                preferred_element_type=jnp.float32) + b1_ref[0]
    h = jnp.where(h >= 0.0, h, 0.01 * h)
    out = jnp.dot(h.astype(jnp.bfloat16), w2_ref[0],
                  preferred_element_type=jnp.float32) + b2_ref[0]
    out_ref[...] = out.astype(jnp.bfloat16)


def _combine_body(eo_ref, flat1_ref, flat2_ref, g1_ref, g2_ref, out_ref):
    slot = jax.lax.broadcasted_iota(jnp.int32, (N // 4, E * C), 1)
    f1 = flat1_ref[...]
    f2 = flat2_ref[...]
    g = (g1_ref[...] * (slot == f1).astype(jnp.float32)
         + g2_ref[...] * (slot == f2).astype(jnp.float32))
    out_ref[...] = jnp.dot(g.astype(jnp.bfloat16), eo_ref[...],
                           preferred_element_type=jnp.float32)


@functools.partial(jax.jit, static_argnames=())
def kernel(x, wg, w1, b1, w2, b2):
    B, S, D = x.shape
    tok = x.reshape(N, D)
    tok_bf = tok.astype(jnp.bfloat16)
    w1_bf = w1.astype(jnp.bfloat16)
    w2_bf = w2.astype(jnp.bfloat16)

    flat1, flat2, g1, g2 = pl.pallas_call(
        _router_body,
        out_shape=(
            jax.ShapeDtypeStruct((N, 1), jnp.int32),
            jax.ShapeDtypeStruct((N, 1), jnp.int32),
            jax.ShapeDtypeStruct((N, 1), jnp.float32),
            jax.ShapeDtypeStruct((N, 1), jnp.float32),
        ),
    )(tok, wg)

    flat1_row = flat1.reshape(1, N)
    flat2_row = flat2.reshape(1, N)

    eo = pl.pallas_call(
        _ffn_body,
        grid=(E,),
        in_specs=[
            pl.BlockSpec((N, D), lambda e: (0, 0)),
            pl.BlockSpec((1, N), lambda e: (0, 0)),
            pl.BlockSpec((1, N), lambda e: (0, 0)),
            pl.BlockSpec((1, D, D_FF), lambda e: (e, 0, 0)),
            pl.BlockSpec((1, 1, D_FF), lambda e: (e, 0, 0)),
            pl.BlockSpec((1, D_FF, D), lambda e: (e, 0, 0)),
            pl.BlockSpec((1, 1, D), lambda e: (e, 0, 0)),
        ],
        out_specs=pl.BlockSpec((C, D), lambda e: (e, 0)),
        out_shape=jax.ShapeDtypeStruct((E * C, D), jnp.bfloat16),
    )(tok_bf, flat1_row, flat2_row, w1_bf, b1.reshape(E, 1, D_FF), w2_bf,
      b2.reshape(E, 1, D))

    out = pl.pallas_call(
        _combine_body,
        grid=(4,),
        in_specs=[
            pl.BlockSpec((E * C, D), lambda t: (0, 0)),
            pl.BlockSpec((N // 4, 1), lambda t: (t, 0)),
            pl.BlockSpec((N // 4, 1), lambda t: (t, 0)),
            pl.BlockSpec((N // 4, 1), lambda t: (t, 0)),
            pl.BlockSpec((N // 4, 1), lambda t: (t, 0)),
        ],
        out_specs=pl.BlockSpec((N // 4, D), lambda t: (t, 0)),
        out_shape=jax.ShapeDtypeStruct((N, D), jnp.float32),
    )(eo, flat1, flat2, g1, g2)

    return out.reshape(B, S, D)


# speedup vs baseline: 1.6572x; 1.6572x over previous
import functools

import jax
import jax.numpy as jnp
from jax.experimental import pallas as pl
from jax.experimental.pallas import tpu as pltpu

E = 8
TOPK = 2
D_MODEL = 1024
D_FF = 4096
N = 2048
C = 512
FF_BLK = 2048
NFF = D_FF // FF_BLK


def _router_body(tok_ref, wg_ref, flat1_ref, flat2_ref, g1_ref, g2_ref):
    tok = tok_ref[...]
    wg = wg_ref[...]
    logits = jnp.dot(tok, wg, preferred_element_type=jnp.float32)
    m = jnp.max(logits, axis=1, keepdims=True)
    ex = jnp.exp(logits - m)
    probs = ex / jnp.sum(ex, axis=1, keepdims=True)

    col = jax.lax.broadcasted_iota(jnp.int32, (N, E), 1)
    big = jnp.int32(E)
    m1 = jnp.max(probs, axis=1, keepdims=True)
    a1 = jnp.min(jnp.where(probs == m1, col, big), axis=1, keepdims=True)
    p2 = jnp.where(col == a1, -1.0, probs)
    m2 = jnp.max(p2, axis=1, keepdims=True)
    a2 = jnp.min(jnp.where(p2 == m2, col, big), axis=1, keepdims=True)

    mask1 = (col == a1).astype(jnp.float32)
    mask2 = (col == a2).astype(jnp.float32)

    ri = jax.lax.broadcasted_iota(jnp.int32, (N, N), 0)
    ci = jax.lax.broadcasted_iota(jnp.int32, (N, N), 1)
    tril = (ci <= ri).astype(jnp.float32)
    cum1 = jnp.dot(tril, mask1, preferred_element_type=jnp.float32)
    cum2 = jnp.dot(tril, mask2, preferred_element_type=jnp.float32)

    pos1 = jnp.sum(cum1 * mask1, axis=1, keepdims=True) - 1.0
    c1 = jnp.sum(mask1, axis=0, keepdims=True)
    pos2 = (jnp.sum(cum2 * mask2, axis=1, keepdims=True) - 1.0
            + jnp.sum(c1 * mask2, axis=1, keepdims=True))
    pos1i = pos1.astype(jnp.int32)
    pos2i = pos2.astype(jnp.int32)

    keep1 = pos1i < C
    keep2 = pos2i < C
    flat1_ref[...] = jnp.where(keep1, a1 * C + pos1i, E * C)
    flat2_ref[...] = jnp.where(keep2, a2 * C + pos2i, E * C)
    g1 = jnp.where(keep1, m1, 0.0)
    g2 = jnp.where(keep2, m2, 0.0)
    denom = g1 + g2 + 1e-9
    g1_ref[...] = g1 / denom
    g2_ref[...] = g2 / denom


def _ffn_body(tok_ref, flat1_ref, flat2_ref, w1_ref, b1_ref, w2_ref, b2_ref,
              out_ref, eb_ref, acc_ref):
    e = pl.program_id(0)
    j = pl.program_id(1)

    @pl.when(j == 0)
    def _dispatch():
        slot = jax.lax.broadcasted_iota(jnp.int32, (C, N), 0) + e * C
        f1 = flat1_ref[...]
        f2 = flat2_ref[...]
        p = ((slot == f1) | (slot == f2)).astype(jnp.bfloat16)
        eb_ref[...] = jnp.dot(p, tok_ref[...],
                              preferred_element_type=jnp.float32
                              ).astype(jnp.bfloat16)

    h = jnp.dot(eb_ref[...], w1_ref[0].astype(jnp.bfloat16),
                preferred_element_type=jnp.float32) + b1_ref[0]
    h = jnp.where(h >= 0.0, h, 0.01 * h)
    part = jnp.dot(h.astype(jnp.bfloat16), w2_ref[0].astype(jnp.bfloat16),
                   preferred_element_type=jnp.float32)

    @pl.when(j == 0)
    def _init():
        acc_ref[...] = part

    @pl.when(j == NFF - 1)
    def _fin():
        out_ref[...] = (acc_ref[...] + part + b2_ref[0]).astype(jnp.bfloat16)


def _combine_body(eo_ref, flat1_ref, flat2_ref, g1_ref, g2_ref, out_ref):
    slot = jax.lax.broadcasted_iota(jnp.int32, (N // 4, E * C), 1)
    f1 = flat1_ref[...]
    f2 = flat2_ref[...]
    g = (g1_ref[...] * (slot == f1).astype(jnp.float32)
         + g2_ref[...] * (slot == f2).astype(jnp.float32))
    out_ref[...] = jnp.dot(g.astype(jnp.bfloat16), eo_ref[...],
                           preferred_element_type=jnp.float32)


@functools.partial(jax.jit, static_argnames=())
def kernel(x, wg, w1, b1, w2, b2):
    B, S, D = x.shape
    tok = x.reshape(N, D)
    tok_bf = tok.astype(jnp.bfloat16)

    flat1, flat2, g1, g2 = pl.pallas_call(
        _router_body,
        out_shape=(
            jax.ShapeDtypeStruct((N, 1), jnp.int32),
            jax.ShapeDtypeStruct((N, 1), jnp.int32),
            jax.ShapeDtypeStruct((N, 1), jnp.float32),
            jax.ShapeDtypeStruct((N, 1), jnp.float32),
        ),
    )(tok, wg)

    flat1_row = flat1.reshape(1, N)
    flat2_row = flat2.reshape(1, N)

    eo = pl.pallas_call(
        _ffn_body,
        grid=(E, NFF),
        in_specs=[
            pl.BlockSpec((N, D), lambda e, j: (0, 0)),
            pl.BlockSpec((1, N), lambda e, j: (0, 0)),
            pl.BlockSpec((1, N), lambda e, j: (0, 0)),
            pl.BlockSpec((1, D, FF_BLK), lambda e, j: (e, 0, j)),
            pl.BlockSpec((1, 1, FF_BLK), lambda e, j: (e, 0, j)),
            pl.BlockSpec((1, FF_BLK, D), lambda e, j: (e, j, 0)),
            pl.BlockSpec((1, 1, D), lambda e, j: (e, 0, 0)),
        ],
        out_specs=pl.BlockSpec((C, D), lambda e, j: (e, 0)),
        out_shape=jax.ShapeDtypeStruct((E * C, D), jnp.bfloat16),
        scratch_shapes=[pltpu.VMEM((C, D), jnp.bfloat16),
                        pltpu.VMEM((C, D), jnp.float32)],
    )(tok_bf, flat1_row, flat2_row, w1, b1.reshape(E, 1, D_FF), w2,
      b2.reshape(E, 1, D))

    out = pl.pallas_call(
        _combine_body,
        grid=(4,),
        in_specs=[
            pl.BlockSpec((E * C, D), lambda t: (0, 0)),
            pl.BlockSpec((N // 4, 1), lambda t: (t, 0)),
            pl.BlockSpec((N // 4, 1), lambda t: (t, 0)),
            pl.BlockSpec((N // 4, 1), lambda t: (t, 0)),
            pl.BlockSpec((N // 4, 1), lambda t: (t, 0)),
        ],
        out_specs=pl.BlockSpec((N // 4, D), lambda t: (t, 0)),
        out_shape=jax.ShapeDtypeStruct((N, D), jnp.float32),
    )(eo, flat1, flat2, g1, g2)

    return out.reshape(B, S, D)
